# baseline (speedup 1.0000x reference)
import jax
import jax.numpy as jnp
from jax.experimental import pallas as pl
from jax.experimental.pallas import tpu as pltpu

_B, _T, _DIN, _DSAE, _K = 64, 8, 256, 16384, 64
_TS = 2048
_NT = _DSAE // _TS
_CS = 512
_NC = _DSAE // _CS
_NBUF = 8
_Q = _TS // _CS


def _fused_body(x_ref, pe_ref, we_ref, b2_ref, wd_hbm, bd_ref,
                z_ref, xhat_ref, loss_ref,
                pre_s, key_s, lo_s, acc_s, wd_v, dsem):
    i = pl.program_id(0)

    def _wd_copy(c):
        slot = jax.lax.rem(c, _NBUF)
        return pltpu.make_async_copy(
            wd_hbm.at[pl.ds(c * _CS, _CS)], wd_v.at[slot], dsem.at[slot])

    @pl.when(i == 0)
    def _preload():
        for c in range(_NBUF):
            _wd_copy(c).start()

    @pl.when(i < _NT)
    def _encode():
        xs = x_ref[:, 0, :]
        for t in range(1, _T):
            xs = xs + x_ref[:, t, :]
        pes = pe_ref[0:1, :]
        for t in range(1, _T):
            pes = pes + pe_ref[t:t + 1, :]
        xs = xs + pes
        pre_t = jnp.dot(xs, we_ref[...], preferred_element_type=jnp.float32) \
            + b2_ref[...]
        bits = jax.lax.bitcast_convert_type(pre_t, jnp.uint32)
        key_t = jnp.where(bits >> 31, ~bits, bits | jnp.uint32(0x80000000))
        for q in range(_Q):
            pre_s[_Q * i + q] = pre_t[:, q * _CS:(q + 1) * _CS]
            key_s[_Q * i + q] = key_t[:, q * _CS:(q + 1) * _CS]

    @pl.when(i == _NT)
    def _bisect():
        def step(it, lo):
            cand = lo | (jnp.uint32(1) << (jnp.uint32(31) - it.astype(jnp.uint32)))
            accs = []
            for j in range(0, _NC, 2):
                a = (key_s[j][:, 0:128] >= cand).astype(jnp.int32)
                for c in range(1, _CS // 128):
                    a = a + (key_s[j][:, c * 128:(c + 1) * 128] >= cand).astype(jnp.int32)
                for c in range(_CS // 128):
                    a = a + (key_s[j + 1][:, c * 128:(c + 1) * 128] >= cand).astype(jnp.int32)
                accs.append(a)
            while len(accs) > 1:
                accs = [accs[p] + accs[p + 1] for p in range(0, len(accs), 2)]
            cnt = jnp.sum(accs[0], axis=1, keepdims=True)
            return jnp.where(cnt >= _K, cand, lo)

        lo = jax.lax.fori_loop(0, 32, step, jnp.zeros((_B, 1), jnp.uint32))
        lo_s[...] = jnp.broadcast_to(lo, (_B, 128))

    @pl.when(i >= _NT)
    def _decode():
        j = i - _NT
        slot = jax.lax.rem(j, _NBUF)
        pre_c = pre_s[j]
        key_c = key_s[j]
        lo = lo_s[:, 0:1]
        zc = jnp.where(key_c >= lo, jnp.maximum(pre_c, 0.0), 0.0)
        z_ref[...] = zc
        _wd_copy(j).wait()
        for t in range(_T):
            ct = jnp.dot(zc, wd_v[slot][:, t, :], preferred_element_type=jnp.float32)

            @pl.when(j == 0)
            def _():
                acc_s[:, t * _DIN:(t + 1) * _DIN] = ct

            @pl.when(j > 0)
            def _():
                acc_s[:, t * _DIN:(t + 1) * _DIN] += ct

        @pl.when(j < _NC - _NBUF)
        def _():
            _wd_copy(j + _NBUF).start()

        @pl.when(j == _NC - 1)
        def _():
            lsum = jnp.zeros((), jnp.float32)
            for t in range(_T):
                xh_t = acc_s[:, t * _DIN:(t + 1) * _DIN] + bd_ref[t:t + 1, :]
                xhat_ref[:, t, :] = xh_t
                d = xh_t - x_ref[:, t, :]
                lsum = lsum + jnp.sum(d * d)
            loss_ref[...] = (lsum / (_B * _T)).reshape(1, 1)


def kernel(x, W_enc, W_dec, b_enc, b_dec, pos_emb):
    b2 = b_enc.reshape(1, _DSAE)

    z, xhat, loss = pl.pallas_call(
        _fused_body,
        grid=(_NT + _NC,),
        in_specs=[
            pl.BlockSpec((_B, _T, _DIN), lambda i: (0, 0, 0)),
            pl.BlockSpec((_T, _DIN), lambda i: (0, 0)),
            pl.BlockSpec((_DIN, _TS), lambda i: (0, jnp.minimum(i, _NT - 1))),
            pl.BlockSpec((1, _TS), lambda i: (0, jnp.minimum(i, _NT - 1))),
            pl.BlockSpec(memory_space=pl.ANY),
            pl.BlockSpec((_T, _DIN), lambda i: (0, 0)),
        ],
        out_specs=[
            pl.BlockSpec((_B, _CS), lambda i: (0, jnp.maximum(i - _NT, 0))),
            pl.BlockSpec((_B, _T, _DIN), lambda i: (0, 0, 0)),
            pl.BlockSpec((1, 1), lambda i: (0, 0)),
        ],
        out_shape=[
            jax.ShapeDtypeStruct((_B, _DSAE), jnp.float32),
            jax.ShapeDtypeStruct((_B, _T, _DIN), jnp.float32),
            jax.ShapeDtypeStruct((1, 1), jnp.float32),
        ],
        scratch_shapes=[
            pltpu.VMEM((_NC, _B, _CS), jnp.float32),
            pltpu.VMEM((_NC, _B, _CS), jnp.uint32),
            pltpu.VMEM((_B, 128), jnp.uint32),
            pltpu.VMEM((_B, _T * _DIN), jnp.float32),
            pltpu.VMEM((_NBUF, _CS, _T, _DIN), jnp.float32),
            pltpu.SemaphoreType.DMA((_NBUF,)),
        ],
    )(x, pos_emb, W_enc, b2, W_dec, b_dec)
    return (loss.reshape(()), xhat, z)


# speedup vs baseline: 1.9440x; 1.9440x over previous
import jax
import jax.numpy as jnp
from jax.experimental import pallas as pl
from jax.experimental.pallas import tpu as pltpu

_B, _T, _DIN, _DSAE, _K = 64, 8, 256, 16384, 64
_TS = 2048
_NT = _DSAE // _TS
_SUB = _TS // 128


def _fused_body(x_ref, pe_ref, we_ref, b2_ref, wd_hbm, bd_ref,
                z_ref, xhat_ref, loss_ref,
                pre_s, key_s, lo_s, acc_s, wd_a, wd_b, dsem):
    i = pl.program_id(0)

    def _wd_copy(c, buf, s):
        return pltpu.make_async_copy(
            wd_hbm.at[pl.ds(c * _TS, _TS)], buf, dsem.at[s])

    @pl.when(i == 0)
    def _preload():
        _wd_copy(0, wd_a, 0).start()
        _wd_copy(1, wd_b, 1).start()

    @pl.when(i < _NT)
    def _encode():
        xs = x_ref[:, 0, :]
        for t in range(1, _T):
            xs = xs + x_ref[:, t, :]
        pes = pe_ref[0:1, :]
        for t in range(1, _T):
            pes = pes + pe_ref[t:t + 1, :]
        xs = xs + pes
        pre_t = jnp.dot(xs, we_ref[...], preferred_element_type=jnp.float32) \
            + b2_ref[...]
        bits = jax.lax.bitcast_convert_type(pre_t, jnp.uint32)
        key_t = jnp.where(bits >> 31, ~bits, bits | jnp.uint32(0x80000000))
        pre_s[i] = pre_t
        key_s[i] = key_t

    @pl.when(i == _NT)
    def _bisect():
        def step(it, lo):
            cand = lo | (jnp.uint32(1) << (jnp.uint32(31) - it.astype(jnp.uint32)))
            accs = []
            for j in range(_NT):
                kj = key_s[j]
                for h in range(2):
                    c0 = h * (_SUB // 2)
                    a = (kj[:, c0 * 128:(c0 + 1) * 128] >= cand).astype(jnp.int32)
                    for c in range(c0 + 1, c0 + _SUB // 2):
                        a = a + (kj[:, c * 128:(c + 1) * 128] >= cand).astype(jnp.int32)
                    accs.append(a)
            while len(accs) > 1:
                accs = [accs[p] + accs[p + 1] for p in range(0, len(accs), 2)]
            cnt = jnp.sum(accs[0], axis=1, keepdims=True)
            return jnp.where(cnt >= _K, cand, lo)

        lo = jax.lax.fori_loop(0, 32, step, jnp.zeros((_B, 1), jnp.uint32))
        lo_s[...] = jnp.broadcast_to(lo, (_B, 128))

    @pl.when(i >= _NT)
    def _decode():
        j = i - _NT
        pre_c = pre_s[j]
        key_c = key_s[j]
        lo = lo_s[:, 0:1]
        zc = jnp.where(key_c >= lo, jnp.maximum(pre_c, 0.0), 0.0)
        z_ref[...] = zc

        def _consume(buf, s):
            _wd_copy(j, buf, s).wait()
            for t in range(_T):
                ct = jnp.dot(zc, buf[:, t, :], preferred_element_type=jnp.float32)

                @pl.when(j == 0)
                def _():
                    acc_s[:, t * _DIN:(t + 1) * _DIN] = ct

                @pl.when(j > 0)
                def _():
                    acc_s[:, t * _DIN:(t + 1) * _DIN] += ct

            @pl.when(j < _NT - 2)
            def _():
                _wd_copy(j + 2, buf, s).start()

        @pl.when(jax.lax.rem(j, 2) == 0)
        def _():
            _consume(wd_a, 0)

        @pl.when(jax.lax.rem(j, 2) == 1)
        def _():
            _consume(wd_b, 1)

        @pl.when(j == _NT - 1)
        def _():
            lsum = jnp.zeros((), jnp.float32)
            for t in range(_T):
                xh_t = acc_s[:, t * _DIN:(t + 1) * _DIN] + bd_ref[t:t + 1, :]
                xhat_ref[:, t, :] = xh_t
                d = xh_t - x_ref[:, t, :]
                lsum = lsum + jnp.sum(d * d)
            loss_ref[...] = (lsum / (_B * _T)).reshape(1, 1)


def kernel(x, W_enc, W_dec, b_enc, b_dec, pos_emb):
    b2 = b_enc.reshape(1, _DSAE)

    z, xhat, loss = pl.pallas_call(
        _fused_body,
        grid=(2 * _NT,),
        in_specs=[
            pl.BlockSpec((_B, _T, _DIN), lambda i: (0, 0, 0)),
            pl.BlockSpec((_T, _DIN), lambda i: (0, 0)),
            pl.BlockSpec((_DIN, _TS), lambda i: (0, jnp.minimum(i, _NT - 1))),
            pl.BlockSpec((1, _TS), lambda i: (0, jnp.minimum(i, _NT - 1))),
            pl.BlockSpec(memory_space=pl.ANY),
            pl.BlockSpec((_T, _DIN), lambda i: (0, 0)),
        ],
        out_specs=[
            pl.BlockSpec((_B, _TS), lambda i: (0, jnp.maximum(i - _NT, 0))),
            pl.BlockSpec((_B, _T, _DIN), lambda i: (0, 0, 0)),
            pl.BlockSpec((1, 1), lambda i: (0, 0)),
        ],
        out_shape=[
            jax.ShapeDtypeStruct((_B, _DSAE), jnp.float32),
            jax.ShapeDtypeStruct((_B, _T, _DIN), jnp.float32),
            jax.ShapeDtypeStruct((1, 1), jnp.float32),
        ],
        scratch_shapes=[
            pltpu.VMEM((_NT, _B, _TS), jnp.float32),
            pltpu.VMEM((_NT, _B, _TS), jnp.uint32),
            pltpu.VMEM((_B, 128), jnp.uint32),
            pltpu.VMEM((_B, _T * _DIN), jnp.float32),
            pltpu.VMEM((_TS, _T, _DIN), jnp.float32),
            pltpu.VMEM((_TS, _T, _DIN), jnp.float32),
            pltpu.SemaphoreType.DMA((2,)),
        ],
    )(x, pos_emb, W_enc, b2, W_dec, b_dec)
    return (loss.reshape(()), xhat, z)


# speedup vs baseline: 2.0640x; 1.0617x over previous
import jax
import jax.numpy as jnp
from jax.experimental import pallas as pl
from jax.experimental.pallas import tpu as pltpu

_B, _T, _DIN, _DSAE, _K = 64, 8, 256, 16384, 64
_TS = 2048
_NT = _DSAE // _TS
_SUB = _TS // 128


def _fused_body(x_ref, pe_ref, we_ref, b2_ref, wd_ref, bd_ref,
                z_ref, xhat_ref, loss_ref,
                pre_s, key_s, lo_s, acc_s):
    i = pl.program_id(0)

    @pl.when(i < _NT)
    def _encode():
        xs = x_ref[:, 0, :]
        for t in range(1, _T):
            xs = xs + x_ref[:, t, :]
        pes = pe_ref[0:1, :]
        for t in range(1, _T):
            pes = pes + pe_ref[t:t + 1, :]
        xs = xs + pes
        pre_t = jnp.dot(xs, we_ref[...], preferred_element_type=jnp.float32) \
            + b2_ref[...]
        bits = jax.lax.bitcast_convert_type(pre_t, jnp.uint32)
        key_t = jnp.where(bits >> 31, ~bits, bits | jnp.uint32(0x80000000))
        pre_s[i] = pre_t
        key_s[i] = key_t

    @pl.when(i == _NT)
    def _bisect():
        def step(it, lo):
            cand = lo | (jnp.uint32(1) << (jnp.uint32(31) - it.astype(jnp.uint32)))
            accs = []
            for j in range(_NT):
                kj = key_s[j]
                for h in range(2):
                    c0 = h * (_SUB // 2)
                    a = (kj[:, c0 * 128:(c0 + 1) * 128] >= cand).astype(jnp.int32)
                    for c in range(c0 + 1, c0 + _SUB // 2):
                        a = a + (kj[:, c * 128:(c + 1) * 128] >= cand).astype(jnp.int32)
                    accs.append(a)
            while len(accs) > 1:
                accs = [accs[p] + accs[p + 1] for p in range(0, len(accs), 2)]
            cnt = jnp.sum(accs[0], axis=1, keepdims=True)
            return jnp.where(cnt >= _K, cand, lo)

        lo = jax.lax.fori_loop(0, 32, step, jnp.zeros((_B, 1), jnp.uint32))
        lo_s[...] = jnp.broadcast_to(lo, (_B, 128))

    @pl.when(i >= _NT)
    def _decode():
        j = i - _NT
        pre_c = pre_s[j]
        key_c = key_s[j]
        lo = lo_s[:, 0:1]
        zc = jnp.where(key_c >= lo, jnp.maximum(pre_c, 0.0), 0.0)
        z_ref[...] = zc
        for t in range(_T):
            ct = jnp.dot(zc, wd_ref[:, t, :], preferred_element_type=jnp.float32)

            @pl.when(j == 0)
            def _():
                acc_s[:, t * _DIN:(t + 1) * _DIN] = ct

            @pl.when(j > 0)
            def _():
                acc_s[:, t * _DIN:(t + 1) * _DIN] += ct

        @pl.when(j == _NT - 1)
        def _():
            lsum = jnp.zeros((), jnp.float32)
            for t in range(_T):
                xh_t = acc_s[:, t * _DIN:(t + 1) * _DIN] + bd_ref[t:t + 1, :]
                xhat_ref[:, t, :] = xh_t
                d = xh_t - x_ref[:, t, :]
                lsum = lsum + jnp.sum(d * d)
            loss_ref[...] = (lsum / (_B * _T)).reshape(1, 1)


def kernel(x, W_enc, W_dec, b_enc, b_dec, pos_emb):
    b2 = b_enc.reshape(1, _DSAE)

    z, xhat, loss = pl.pallas_call(
        _fused_body,
        grid=(2 * _NT,),
        in_specs=[
            pl.BlockSpec((_B, _T, _DIN), lambda i: (0, 0, 0)),
            pl.BlockSpec((_T, _DIN), lambda i: (0, 0)),
            pl.BlockSpec((_DIN, _TS), lambda i: (0, jnp.minimum(i, _NT - 1))),
            pl.BlockSpec((1, _TS), lambda i: (0, jnp.minimum(i, _NT - 1))),
            pl.BlockSpec((_TS, _T, _DIN), lambda i: (jnp.maximum(i - _NT, 0), 0, 0)),
            pl.BlockSpec((_T, _DIN), lambda i: (0, 0)),
        ],
        out_specs=[
            pl.BlockSpec((_B, _TS), lambda i: (0, jnp.maximum(i - _NT, 0))),
            pl.BlockSpec((_B, _T, _DIN), lambda i: (0, 0, 0)),
            pl.BlockSpec((1, 1), lambda i: (0, 0)),
        ],
        out_shape=[
            jax.ShapeDtypeStruct((_B, _DSAE), jnp.float32),
            jax.ShapeDtypeStruct((_B, _T, _DIN), jnp.float32),
            jax.ShapeDtypeStruct((1, 1), jnp.float32),
        ],
        scratch_shapes=[
            pltpu.VMEM((_NT, _B, _TS), jnp.float32),
            pltpu.VMEM((_NT, _B, _TS), jnp.uint32),
            pltpu.VMEM((_B, 128), jnp.uint32),
            pltpu.VMEM((_B, _T * _DIN), jnp.float32),
        ],
    )(x, pos_emb, W_enc, b2, W_dec, b_dec)
    return (loss.reshape(()), xhat, z)


# speedup vs baseline: 2.2006x; 1.0662x over previous
import jax
import jax.numpy as jnp
from jax.experimental import pallas as pl
from jax.experimental.pallas import tpu as pltpu

_B, _T, _DIN, _DSAE, _K = 64, 8, 256, 16384, 64
_TS = 2048
_NT = _DSAE // _TS
_SUB = _TS // 128


def _fused_body(x_ref, pe_ref, we_ref, b2_ref, wd_ref, bd_ref,
                z_ref, xhat_ref, loss_ref,
                pre_s, kh_s, kl_s, lo_s, acc_s):
    i = pl.program_id(0)

    @pl.when(i < _NT)
    def _encode():
        xs = x_ref[:, 0, :]
        for t in range(1, _T):
            xs = xs + x_ref[:, t, :]
        pes = pe_ref[0:1, :]
        for t in range(1, _T):
            pes = pes + pe_ref[t:t + 1, :]
        xs = xs + pes
        pre_t = jnp.dot(xs, we_ref[...], preferred_element_type=jnp.float32) \
            + b2_ref[...]
        bits = jax.lax.bitcast_convert_type(pre_t, jnp.uint32)
        key_t = jnp.where(bits >> 31, ~bits, bits | jnp.uint32(0x80000000))
        pre_s[i] = pre_t
        kh_s[i] = ((key_t >> 16) ^ jnp.uint32(0x8000)).astype(jnp.int16)
        kl_s[i] = ((key_t & jnp.uint32(0xFFFF)) ^ jnp.uint32(0x8000)).astype(jnp.int16)

    @pl.when(i == _NT)
    def _bisect():
        def count16(src, cand_b, cmp):
            accs = []
            for j in range(_NT):
                kj = src[j]
                for h in range(2):
                    c0 = h * (_SUB // 4)
                    sl = kj[:, c0 * 256:(c0 + 1) * 256]
                    a = cmp(sl, cand_b).astype(jnp.int16)
                    for c in range(c0 + 1, c0 + _SUB // 4):
                        sl = kj[:, c * 256:(c + 1) * 256]
                        a = a + cmp(sl, cand_b).astype(jnp.int16)
                    accs.append(a)
            while len(accs) > 1:
                accs = [accs[p] + accs[p + 1] for p in range(0, len(accs), 2)]
            return jnp.sum(accs[0].astype(jnp.int32), axis=1, keepdims=True)

        def bias16(u):
            return (u ^ 0x8000).astype(jnp.int16)

        def phase(src, need):
            def step(it, lo):
                cand = lo | (1 << (15 - it))
                cnt = count16(src, bias16(cand), lambda s, c: s >= c)
                return jnp.where(cnt >= need, cand, lo)
            return jax.lax.fori_loop(0, 16, step, jnp.zeros((_B, 1), jnp.int32))

        hi = phase(kh_s, _K)
        hi_b = bias16(hi)
        c_gt = count16(kh_s, hi_b, lambda s, c: s > c)
        for j in range(_NT):
            kl_s[j] = jnp.where(kh_s[j] == hi_b, kl_s[j], jnp.int16(-0x8000))
        lo16 = phase(kl_s, _K - c_gt)

        tkey = (hi.astype(jnp.uint32) << 16) | lo16.astype(jnp.uint32)
        tbits = jnp.where(tkey >> 31, tkey ^ jnp.uint32(0x80000000), ~tkey)
        thr = jax.lax.bitcast_convert_type(tbits, jnp.float32)
        lo_s[...] = jnp.broadcast_to(thr, (_B, 128))

    @pl.when(i >= _NT)
    def _decode():
        j = i - _NT
        pre_c = pre_s[j]
        thr = lo_s[:, 0:1]
        zc = jnp.where(pre_c >= thr, jnp.maximum(pre_c, 0.0), 0.0)
        z_ref[...] = zc
        for t in range(_T):
            ct = jnp.dot(zc, wd_ref[:, t, :], preferred_element_type=jnp.float32)

            @pl.when(j == 0)
            def _():
                acc_s[:, t * _DIN:(t + 1) * _DIN] = ct

            @pl.when(j > 0)
            def _():
                acc_s[:, t * _DIN:(t + 1) * _DIN] += ct

        @pl.when(j == _NT - 1)
        def _():
            lsum = jnp.zeros((), jnp.float32)
            for t in range(_T):
                xh_t = acc_s[:, t * _DIN:(t + 1) * _DIN] + bd_ref[t:t + 1, :]
                xhat_ref[:, t, :] = xh_t
                d = xh_t - x_ref[:, t, :]
                lsum = lsum + jnp.sum(d * d)
            loss_ref[...] = (lsum / (_B * _T)).reshape(1, 1)


def kernel(x, W_enc, W_dec, b_enc, b_dec, pos_emb):
    b2 = b_enc.reshape(1, _DSAE)

    z, xhat, loss = pl.pallas_call(
        _fused_body,
        grid=(2 * _NT,),
        in_specs=[
            pl.BlockSpec((_B, _T, _DIN), lambda i: (0, 0, 0)),
            pl.BlockSpec((_T, _DIN), lambda i: (0, 0)),
            pl.BlockSpec((_DIN, _TS), lambda i: (0, jnp.minimum(i, _NT - 1))),
            pl.BlockSpec((1, _TS), lambda i: (0, jnp.minimum(i, _NT - 1))),
            pl.BlockSpec((_TS, _T, _DIN), lambda i: (jnp.maximum(i - _NT, 0), 0, 0)),
            pl.BlockSpec((_T, _DIN), lambda i: (0, 0)),
        ],
        out_specs=[
            pl.BlockSpec((_B, _TS), lambda i: (0, jnp.maximum(i - _NT, 0))),
            pl.BlockSpec((_B, _T, _DIN), lambda i: (0, 0, 0)),
            pl.BlockSpec((1, 1), lambda i: (0, 0)),
        ],
        out_shape=[
            jax.ShapeDtypeStruct((_B, _DSAE), jnp.float32),
            jax.ShapeDtypeStruct((_B, _T, _DIN), jnp.float32),
            jax.ShapeDtypeStruct((1, 1), jnp.float32),
        ],
        scratch_shapes=[
            pltpu.VMEM((_NT, _B, _TS), jnp.float32),
            pltpu.VMEM((_NT, _B, _TS), jnp.int16),
            pltpu.VMEM((_NT, _B, _TS), jnp.int16),
            pltpu.VMEM((_B, 128), jnp.float32),
            pltpu.VMEM((_B, _T * _DIN), jnp.float32),
        ],
    )(x, pos_emb, W_enc, b2, W_dec, b_dec)
    return (loss.reshape(()), xhat, z)
